# baseline (speedup 1.0000x reference)
import functools

import jax
import jax.numpy as jnp
from jax import lax
from jax.experimental import pallas as pl
from jax.experimental.pallas import tpu as pltpu
from jax.experimental.pallas import tpu_sc as plsc

_NUM = 1000000
_DIM = 32
_BATCH = 16384
_FIELDS = 26
_B = _BATCH * _FIELDS
_NC = 2
_NS = 16
_NW = _NC * _NS
_BPW = _B // _NW
_CHUNK = 1024
_NCHUNK = _BPW // _CHUNK
_NBUF = 3

_TCOLS = 8192
_TQ = _TCOLS // 4
_TGRID = (_NUM + _TCOLS - 1) // _TCOLS


def _detranspose_body(t_ref, o_ref):
    t = t_ref[...]
    eye = jnp.eye(_DIM, dtype=jnp.float32)
    blk = jnp.concatenate(
        [lax.dot_general(t[:, q * _TQ:(q + 1) * _TQ], eye,
                         (((0,), (0,)), ((), ())),
                         precision=lax.Precision.HIGHEST)
         for q in range(4)], axis=1)
    o_ref[...] = blk.reshape(_TQ * 128)


_detranspose = pl.pallas_call(
    _detranspose_body,
    grid=(_TGRID,),
    in_specs=[pl.BlockSpec((_DIM, _TCOLS), lambda j: (0, j))],
    out_specs=pl.BlockSpec((_TQ * 128,), lambda j: (j,)),
    out_shape=jax.ShapeDtypeStruct((_TGRID * _TQ * 128,), jnp.float32),
)

_mesh = plsc.VectorSubcoreMesh(core_axis_name="c", subcore_axis_name="s")


@functools.partial(
    pl.kernel,
    mesh=_mesh,
    out_type=jax.ShapeDtypeStruct((_B, _DIM), jnp.float32),
    scratch_types=[
        pltpu.VMEM((_BPW,), jnp.int32),
        *[pltpu.VMEM((_CHUNK, _DIM), jnp.float32) for _ in range(_NBUF)],
        pltpu.SemaphoreType.DMA,
        *[pltpu.SemaphoreType.DMA for _ in range(_NBUF)],
        *[pltpu.SemaphoreType.DMA for _ in range(_NBUF)],
    ],
    compiler_params=pltpu.CompilerParams(use_tc_tiling_on_sc=False),
)
def _gather_kernel(idx_hbm, table_hbm, out_hbm, idx_v, *rest):
    rows = rest[:_NBUF]
    sem_i = rest[_NBUF]
    sem_g = rest[_NBUF + 1:2 * _NBUF + 1]
    sem_w = rest[2 * _NBUF + 1:]

    wid = lax.axis_index("s") * _NC + lax.axis_index("c")
    base = wid * _BPW
    pltpu.async_copy(idx_hbm.at[pl.ds(base, _BPW)], idx_v, sem_i).wait()

    gathers = [None] * _NCHUNK
    writes = [None] * _NCHUNK

    def fire_gather(i):
        b = i % _NBUF
        gathers[i] = pltpu.async_copy(
            table_hbm.at[idx_v.at[pl.ds(i * _CHUNK, _CHUNK)]],
            rows[b], sem_g[b])

    def fire_write(j):
        b = j % _NBUF
        writes[j] = pltpu.async_copy(
            rows[b], out_hbm.at[pl.ds(base + j * _CHUNK, _CHUNK)], sem_w[b])

    for i in range(_NCHUNK):
        if i >= _NBUF:
            writes[i - _NBUF].wait()
        fire_gather(i)
        if i >= _NBUF - 1:
            j = i - (_NBUF - 1)
            gathers[j].wait()
            fire_write(j)
    for j in range(_NCHUNK - (_NBUF - 1), _NCHUNK):
        gathers[j].wait()
        fire_write(j)
    for j in range(_NCHUNK - _NBUF, _NCHUNK):
        writes[j].wait()


_OQ = _BATCH // 4


def _out_transpose_body(g_ref, o_ref):
    g = g_ref[...].reshape(_OQ, 128)
    eye = jnp.eye(_DIM, dtype=jnp.float32)
    blk = jnp.concatenate(
        [lax.dot_general(eye, g[:, q * _DIM:(q + 1) * _DIM],
                         (((1,), (1,)), ((), ())),
                         precision=lax.Precision.HIGHEST)
         for q in range(4)], axis=1)
    o_ref[...] = blk.reshape(1, _DIM, _BATCH)


_out_transpose = pl.pallas_call(
    _out_transpose_body,
    grid=(_FIELDS,),
    in_specs=[pl.BlockSpec((_OQ * 128,), lambda f: (f,))],
    out_specs=pl.BlockSpec((1, _DIM, _BATCH), lambda f: (f, 0, 0)),
    out_shape=jax.ShapeDtypeStruct((_FIELDS, _DIM, _BATCH), jnp.float32),
)


def kernel(token_ids, embeddings):
    table_lin = _detranspose(embeddings.T).reshape(-1, _DIM)

    ids = token_ids.astype(jnp.int32)
    j, u = ids // _TCOLS, ids % _TCOLS
    ids = 4 * (j * _TQ + u % _TQ) + u // _TQ

    idx = ids.T.reshape(_FIELDS, 4, _OQ).swapaxes(1, 2).reshape(-1)

    g = _gather_kernel(idx, table_lin)
    o2 = _out_transpose(g.reshape(-1))
    return o2.transpose(2, 0, 1)


# speedup vs baseline: 3.1596x; 3.1596x over previous
import functools

import jax
import jax.numpy as jnp
from jax import lax
from jax.experimental import pallas as pl
from jax.experimental.pallas import tpu as pltpu
from jax.experimental.pallas import tpu_sc as plsc

_NUM = 1000000
_DIM = 32
_BATCH = 16384
_FIELDS = 26
_B = _BATCH * _FIELDS
_NC = 2
_NS = 16
_NW = _NC * _NS
_BPW = _B // _NW
_CHUNK = 1024
_NCHUNK = _BPW // _CHUNK
_NBUF = 3

_TCOLS = 16384
_TQ = _TCOLS // 4
_TGRID = (_NUM + _TCOLS - 1) // _TCOLS


def _detranspose_body(t_ref, o_ref):
    t = t_ref[...]
    m = jnp.concatenate(
        [t[:, q * _TQ:(q + 1) * _TQ] for q in range(4)], axis=0)
    o_ref[...] = m.T.reshape(_TQ * 128)


_detranspose = pl.pallas_call(
    _detranspose_body,
    grid=(_TGRID,),
    in_specs=[pl.BlockSpec((_DIM, _TCOLS), lambda j: (0, j))],
    out_specs=pl.BlockSpec((_TQ * 128,), lambda j: (j,)),
    out_shape=jax.ShapeDtypeStruct((_TGRID * _TQ * 128,), jnp.float32),
)

_mesh = plsc.VectorSubcoreMesh(core_axis_name="c", subcore_axis_name="s")


@functools.partial(
    pl.kernel,
    mesh=_mesh,
    out_type=jax.ShapeDtypeStruct((_B, _DIM), jnp.float32),
    scratch_types=[
        pltpu.VMEM((_BPW,), jnp.int32),
        *[pltpu.VMEM((_CHUNK, _DIM), jnp.float32) for _ in range(_NBUF)],
        pltpu.SemaphoreType.DMA,
        *[pltpu.SemaphoreType.DMA for _ in range(_NBUF)],
        *[pltpu.SemaphoreType.DMA for _ in range(_NBUF)],
    ],
    compiler_params=pltpu.CompilerParams(use_tc_tiling_on_sc=False),
)
def _gather_kernel(idx_hbm, table_hbm, out_hbm, idx_v, *rest):
    rows = rest[:_NBUF]
    sem_i = rest[_NBUF]
    sem_g = rest[_NBUF + 1:2 * _NBUF + 1]
    sem_w = rest[2 * _NBUF + 1:]

    wid = lax.axis_index("s") * _NC + lax.axis_index("c")
    base = wid * _BPW
    pltpu.async_copy(idx_hbm.at[pl.ds(base, _BPW)], idx_v, sem_i).wait()

    gathers = [None] * _NCHUNK
    writes = [None] * _NCHUNK

    def fire_gather(i):
        b = i % _NBUF
        gathers[i] = pltpu.async_copy(
            table_hbm.at[idx_v.at[pl.ds(i * _CHUNK, _CHUNK)]],
            rows[b], sem_g[b])

    def fire_write(j):
        b = j % _NBUF
        writes[j] = pltpu.async_copy(
            rows[b], out_hbm.at[pl.ds(base + j * _CHUNK, _CHUNK)], sem_w[b])

    for i in range(_NCHUNK):
        if i >= _NBUF:
            writes[i - _NBUF].wait()
        fire_gather(i)
        if i >= _NBUF - 1:
            j = i - (_NBUF - 1)
            gathers[j].wait()
            fire_write(j)
    for j in range(_NCHUNK - (_NBUF - 1), _NCHUNK):
        gathers[j].wait()
        fire_write(j)
    for j in range(_NCHUNK - _NBUF, _NCHUNK):
        writes[j].wait()


_OQ = _BATCH // 4


def _out_transpose_body(g_ref, o_ref):
    g = g_ref[...].reshape(_OQ, 128)
    parts = g.T.reshape(4, _DIM, _OQ)
    blk = jnp.concatenate([parts[q] for q in range(4)], axis=1)
    o_ref[...] = blk.reshape(1, _DIM, _BATCH)


_out_transpose = pl.pallas_call(
    _out_transpose_body,
    grid=(_FIELDS,),
    in_specs=[pl.BlockSpec((_OQ * 128,), lambda f: (f,))],
    out_specs=pl.BlockSpec((1, _DIM, _BATCH), lambda f: (f, 0, 0)),
    out_shape=jax.ShapeDtypeStruct((_FIELDS, _DIM, _BATCH), jnp.float32),
)


def kernel(token_ids, embeddings):
    table_lin = _detranspose(embeddings.T).reshape(-1, _DIM)

    ids = token_ids.astype(jnp.int32)
    j, u = ids // _TCOLS, ids % _TCOLS
    ids = 4 * (j * _TQ + u % _TQ) + u // _TQ

    idx = ids.T.reshape(_FIELDS, 4, _OQ).swapaxes(1, 2).reshape(-1)

    g = _gather_kernel(idx, table_lin)
    o2 = _out_transpose(g.reshape(-1))
    return o2.transpose(2, 0, 1)


# speedup vs baseline: 3.3299x; 1.0539x over previous
import functools

import jax
import jax.numpy as jnp
from jax import lax
from jax.experimental import pallas as pl
from jax.experimental.pallas import tpu as pltpu
from jax.experimental.pallas import tpu_sc as plsc

_NUM = 1000000
_DIM = 32
_BATCH = 16384
_FIELDS = 26
_B = _BATCH * _FIELDS
_NC = 2
_NS = 16
_NW = _NC * _NS
_BPW = _B // _NW
_CHUNK = 1024
_NCHUNK = _BPW // _CHUNK
_NBUF = 3

_TCOLS = 32768
_TQ = _TCOLS // 4
_TGRID = (_NUM + _TCOLS - 1) // _TCOLS


def _detranspose_body(t_ref, o_ref):
    t = t_ref[...]
    m = jnp.concatenate(
        [t[:, q * _TQ:(q + 1) * _TQ] for q in range(4)], axis=0)
    o_ref[...] = m.T.reshape(_TQ * 128)


_detranspose = pl.pallas_call(
    _detranspose_body,
    grid=(_TGRID,),
    in_specs=[pl.BlockSpec((_DIM, _TCOLS), lambda j: (0, j))],
    out_specs=pl.BlockSpec((_TQ * 128,), lambda j: (j,)),
    out_shape=jax.ShapeDtypeStruct((_TGRID * _TQ * 128,), jnp.float32),
)

_mesh = plsc.VectorSubcoreMesh(core_axis_name="c", subcore_axis_name="s")


@functools.partial(
    pl.kernel,
    mesh=_mesh,
    out_type=jax.ShapeDtypeStruct((_B, _DIM), jnp.float32),
    scratch_types=[
        pltpu.VMEM((_BPW,), jnp.int32),
        *[pltpu.VMEM((_CHUNK, _DIM), jnp.float32) for _ in range(_NBUF)],
        pltpu.SemaphoreType.DMA,
        *[pltpu.SemaphoreType.DMA for _ in range(_NBUF)],
        *[pltpu.SemaphoreType.DMA for _ in range(_NBUF)],
    ],
    compiler_params=pltpu.CompilerParams(use_tc_tiling_on_sc=False),
)
def _gather_kernel(idx_hbm, table_hbm, out_hbm, idx_v, *rest):
    rows = rest[:_NBUF]
    sem_i = rest[_NBUF]
    sem_g = rest[_NBUF + 1:2 * _NBUF + 1]
    sem_w = rest[2 * _NBUF + 1:]

    wid = lax.axis_index("s") * _NC + lax.axis_index("c")
    base = wid * _BPW
    pltpu.async_copy(idx_hbm.at[pl.ds(base, _BPW)], idx_v, sem_i).wait()

    gathers = [None] * _NCHUNK
    writes = [None] * _NCHUNK

    def fire_gather(i):
        b = i % _NBUF
        gathers[i] = pltpu.async_copy(
            table_hbm.at[idx_v.at[pl.ds(i * _CHUNK, _CHUNK)]],
            rows[b], sem_g[b])

    def fire_write(j):
        b = j % _NBUF
        writes[j] = pltpu.async_copy(
            rows[b], out_hbm.at[pl.ds(base + j * _CHUNK, _CHUNK)], sem_w[b])

    for i in range(_NCHUNK):
        if i >= _NBUF:
            writes[i - _NBUF].wait()
        fire_gather(i)
        if i >= _NBUF - 1:
            j = i - (_NBUF - 1)
            gathers[j].wait()
            fire_write(j)
    for j in range(_NCHUNK - (_NBUF - 1), _NCHUNK):
        gathers[j].wait()
        fire_write(j)
    for j in range(_NCHUNK - _NBUF, _NCHUNK):
        writes[j].wait()


_OQ = _BATCH // 4


def _out_transpose_body(g_ref, o_ref):
    g = g_ref[...].reshape(_OQ, 128)
    parts = g.T.reshape(4, _DIM, _OQ)
    blk = jnp.concatenate([parts[q] for q in range(4)], axis=1)
    o_ref[...] = blk.reshape(1, _DIM, _BATCH)


_out_transpose = pl.pallas_call(
    _out_transpose_body,
    grid=(_FIELDS,),
    in_specs=[pl.BlockSpec((_OQ * 128,), lambda f: (f,))],
    out_specs=pl.BlockSpec((1, _DIM, _BATCH), lambda f: (f, 0, 0)),
    out_shape=jax.ShapeDtypeStruct((_FIELDS, _DIM, _BATCH), jnp.float32),
)


def kernel(token_ids, embeddings):
    table_lin = _detranspose(embeddings.T).reshape(-1, _DIM)

    ids = token_ids.astype(jnp.int32)
    j, u = ids // _TCOLS, ids % _TCOLS
    ids = 4 * (j * _TQ + u % _TQ) + u // _TQ

    idx = ids.T.reshape(_FIELDS, 4, _OQ).swapaxes(1, 2).reshape(-1)

    g = _gather_kernel(idx, table_lin)
    o2 = _out_transpose(g.reshape(-1))
    return o2.transpose(2, 0, 1)


# speedup vs baseline: 3.3524x; 1.0068x over previous
import functools

import jax
import jax.numpy as jnp
from jax import lax
from jax.experimental import pallas as pl
from jax.experimental.pallas import tpu as pltpu
from jax.experimental.pallas import tpu_sc as plsc

_NUM = 1000000
_DIM = 32
_BATCH = 16384
_FIELDS = 26
_B = _BATCH * _FIELDS
_NC = 2
_NS = 16
_NW = _NC * _NS
_BPW = _B // _NW
_CHUNK = 1024
_NCHUNK = _BPW // _CHUNK
_NBUF = 3

_TCOLS = 65536
_TQ = _TCOLS // 4
_TGRID = (_NUM + _TCOLS - 1) // _TCOLS


def _detranspose_body(t_ref, o_ref):
    t = t_ref[...]
    m = jnp.concatenate(
        [t[:, q * _TQ:(q + 1) * _TQ] for q in range(4)], axis=0)
    o_ref[...] = m.T.reshape(_TQ * 128)


_detranspose = pl.pallas_call(
    _detranspose_body,
    grid=(_TGRID,),
    in_specs=[pl.BlockSpec((_DIM, _TCOLS), lambda j: (0, j))],
    out_specs=pl.BlockSpec((_TQ * 128,), lambda j: (j,)),
    out_shape=jax.ShapeDtypeStruct((_TGRID * _TQ * 128,), jnp.float32),
)

_mesh = plsc.VectorSubcoreMesh(core_axis_name="c", subcore_axis_name="s")


@functools.partial(
    pl.kernel,
    mesh=_mesh,
    out_type=jax.ShapeDtypeStruct((_B, _DIM), jnp.float32),
    scratch_types=[
        pltpu.VMEM((_BPW,), jnp.int32),
        *[pltpu.VMEM((_CHUNK, _DIM), jnp.float32) for _ in range(_NBUF)],
        pltpu.SemaphoreType.DMA,
        *[pltpu.SemaphoreType.DMA for _ in range(_NBUF)],
        *[pltpu.SemaphoreType.DMA for _ in range(_NBUF)],
    ],
    compiler_params=pltpu.CompilerParams(use_tc_tiling_on_sc=False),
)
def _gather_kernel(idx_hbm, table_hbm, out_hbm, idx_v, *rest):
    rows = rest[:_NBUF]
    sem_i = rest[_NBUF]
    sem_g = rest[_NBUF + 1:2 * _NBUF + 1]
    sem_w = rest[2 * _NBUF + 1:]

    wid = lax.axis_index("s") * _NC + lax.axis_index("c")
    base = wid * _BPW
    pltpu.async_copy(idx_hbm.at[pl.ds(base, _BPW)], idx_v, sem_i).wait()

    gathers = [None] * _NCHUNK
    writes = [None] * _NCHUNK

    def fire_gather(i):
        b = i % _NBUF
        gathers[i] = pltpu.async_copy(
            table_hbm.at[idx_v.at[pl.ds(i * _CHUNK, _CHUNK)]],
            rows[b], sem_g[b])

    def fire_write(j):
        b = j % _NBUF
        writes[j] = pltpu.async_copy(
            rows[b], out_hbm.at[pl.ds(base + j * _CHUNK, _CHUNK)], sem_w[b])

    for i in range(_NCHUNK):
        if i >= _NBUF:
            writes[i - _NBUF].wait()
        fire_gather(i)
        if i >= _NBUF - 1:
            j = i - (_NBUF - 1)
            gathers[j].wait()
            fire_write(j)
    for j in range(_NCHUNK - (_NBUF - 1), _NCHUNK):
        gathers[j].wait()
        fire_write(j)
    for j in range(_NCHUNK - _NBUF, _NCHUNK):
        writes[j].wait()


_OQ = _BATCH // 4


def _out_transpose_body(g_ref, o_ref):
    g = g_ref[...].reshape(_OQ, 128)
    parts = g.T.reshape(4, _DIM, _OQ)
    blk = jnp.concatenate([parts[q] for q in range(4)], axis=1)
    o_ref[...] = blk.reshape(1, _DIM, _BATCH)


_out_transpose = pl.pallas_call(
    _out_transpose_body,
    grid=(_FIELDS,),
    in_specs=[pl.BlockSpec((_OQ * 128,), lambda f: (f,))],
    out_specs=pl.BlockSpec((1, _DIM, _BATCH), lambda f: (f, 0, 0)),
    out_shape=jax.ShapeDtypeStruct((_FIELDS, _DIM, _BATCH), jnp.float32),
)


def kernel(token_ids, embeddings):
    table_lin = _detranspose(embeddings.T).reshape(-1, _DIM)

    ids = token_ids.astype(jnp.int32)
    j, u = ids // _TCOLS, ids % _TCOLS
    ids = 4 * (j * _TQ + u % _TQ) + u // _TQ

    idx = ids.T.reshape(_FIELDS, 4, _OQ).swapaxes(1, 2).reshape(-1)

    g = _gather_kernel(idx, table_lin)
    o2 = _out_transpose(g.reshape(-1))
    return o2.transpose(2, 0, 1)
